# baseline (speedup 1.0000x reference)
import functools

import jax
import jax.numpy as jnp
from jax import lax
from jax.experimental import pallas as pl
from jax.experimental.pallas import tpu as pltpu
from jax.experimental.pallas import tpu_sc as plsc

S = 256
A = 256
W = 8
D = 64
AW = A * W
SW = S * W

_NC, _NS = 2, 16
_NW = _NC * _NS
_BPW = (AW + SW) // _NW


def _gather_body(idx_hbm, none_idx_hbm, table_hbm, out_hbm, none_out_hbm,
                 idx_v, rows_v, nidx_v, nrow_v, sem):
    wid = lax.axis_index("s") * _NC + lax.axis_index("c")
    base = wid * _BPW
    pltpu.sync_copy(idx_hbm.at[pl.ds(base, _BPW)], idx_v)
    pltpu.async_copy(table_hbm.at[idx_v], rows_v, sem).wait()
    pltpu.sync_copy(rows_v, out_hbm.at[pl.ds(base, _BPW)])

    @pl.when(wid == 0)
    def _():
        pltpu.sync_copy(none_idx_hbm, nidx_v)
        pltpu.async_copy(table_hbm.at[nidx_v], nrow_v, sem).wait()
        pltpu.sync_copy(nrow_v, none_out_hbm)


@functools.cache
def _get_gather():
    return pl.kernel(
        _gather_body,
        mesh=plsc.VectorSubcoreMesh(core_axis_name="c", subcore_axis_name="s",
                                    num_cores=_NC, num_subcores=_NS),
        out_type=[
            jax.ShapeDtypeStruct((AW + SW, D), jnp.float32),
            jax.ShapeDtypeStruct((1, D), jnp.float32),
        ],
        scratch_types=[
            pltpu.VMEM((_BPW,), jnp.int32),
            pltpu.VMEM((_BPW, D), jnp.float32),
            pltpu.VMEM((1,), jnp.int32),
            pltpu.VMEM((1, D), jnp.float32),
            pltpu.SemaphoreType.DMA,
        ],
        compiler_params=pltpu.CompilerParams(use_tc_tiling_on_sc=False),
    )


def _dense_body(g_ref, none_ref, sims_ref, swe_ref):
    f32 = jnp.float32
    none_row = none_ref[:]
    ones_row = jnp.ones((1, D), f32)
    ann = g_ref[0:AW, :]
    an_row = jnp.sqrt(lax.dot_general(
        ones_row, ann * ann, (((1,), (1,)), ((), ())),
        preferred_element_type=f32))
    m = jnp.zeros((S, A), f32)
    swe = jnp.zeros((S, D), f32)
    for ws in range(W):
        sen_ws = g_ref[AW + ws * S:AW + (ws + 1) * S, :]
        swe = swe + sen_ws
        sn_c = jnp.sqrt(jnp.sum(sen_ws * sen_ws, axis=1, keepdims=True))
        eqcnt = jnp.sum(jnp.where(sen_ws == none_row, 1.0, 0.0),
                        axis=1, keepdims=True)
        isnone_c = eqcnt == f32(D)
        d_ws = lax.dot_general(sen_ws, ann, (((1,), (1,)), ((), ())),
                               preferred_element_type=f32)
        for wa in range(W):
            dvals = d_ws[:, wa * A:(wa + 1) * A]
            an_r = an_row[:, wa * A:(wa + 1) * A]
            denom = jnp.maximum(sn_c * an_r, 1e-8)
            sval = dvals / denom
            sval = jnp.where(isnone_c, 0.0, sval)
            m = jnp.where((sval >= m) | (sval < 0.0), sval, m)
    sims_ref[:] = m
    swe_ref[:] = swe


_dense = pl.pallas_call(
    _dense_body,
    out_shape=[
        jax.ShapeDtypeStruct((S, A), jnp.float32),
        jax.ShapeDtypeStruct((S, D), jnp.float32),
    ],
)


def kernel(ann_cats, sen_cats, none_idx, table):
    ann_t = ann_cats.astype(jnp.int32).T.reshape(-1)
    sen_t = sen_cats.astype(jnp.int32).T.reshape(-1)
    idx = jnp.concatenate([ann_t, sen_t], axis=0)
    g, none_row = _get_gather()(idx, none_idx.astype(jnp.int32), table)
    max_sims, swe = _dense(g, none_row)
    return max_sims, swe


# speedup vs baseline: 1.7072x; 1.7072x over previous
import functools

import jax
import jax.numpy as jnp
from jax import lax
from jax.experimental import pallas as pl
from jax.experimental.pallas import tpu as pltpu
from jax.experimental.pallas import tpu_sc as plsc

S = 256
A = 256
W = 8
D = 64
AW = A * W
SW = S * W

_NC, _NS = 2, 16
_NW = _NC * _NS
_BPW = (AW + SW) // _NW


def _gather_body(idx_hbm, none_idx_hbm, table_hbm, out_hbm, none_out_hbm,
                 idx_s, nidx_s, rows_v, sem):
    wid = lax.axis_index("s") * _NC + lax.axis_index("c")
    base = wid * _BPW
    pltpu.sync_copy(idx_hbm.at[pl.ds(base, _BPW)], idx_s)
    copies = []
    for c0 in range(0, _BPW, 16):
        vec = idx_s[pl.ds(c0, 16)]
        for l in range(16):
            copies.append(pltpu.async_copy(
                table_hbm.at[pl.ds(vec[l], 1)],
                rows_v.at[pl.ds(c0 + l, 1)], sem))
    for c in copies:
        c.wait()
    pltpu.sync_copy(rows_v, out_hbm.at[pl.ds(base, _BPW)])

    @pl.when(wid == 0)
    def _():
        pltpu.sync_copy(none_idx_hbm, nidx_s.at[pl.ds(0, 1)])
        nvec = nidx_s[...]
        pltpu.async_copy(table_hbm.at[pl.ds(nvec[0], 1)],
                         rows_v.at[pl.ds(0, 1)], sem).wait()
        pltpu.sync_copy(rows_v.at[pl.ds(0, 1)], none_out_hbm)


@functools.cache
def _get_gather():
    return pl.kernel(
        _gather_body,
        mesh=plsc.VectorSubcoreMesh(core_axis_name="c", subcore_axis_name="s",
                                    num_cores=_NC, num_subcores=_NS),
        out_type=[
            jax.ShapeDtypeStruct((AW + SW, D), jnp.float32),
            jax.ShapeDtypeStruct((1, D), jnp.float32),
        ],
        scratch_types=[
            pltpu.VMEM((_BPW,), jnp.int32),
            pltpu.VMEM((16,), jnp.int32),
            pltpu.VMEM((_BPW, D), jnp.float32),
            pltpu.SemaphoreType.DMA,
        ],
    )


def _dense_body(g_ref, none_ref, sims_ref, swe_ref):
    f32 = jnp.float32
    none_row = none_ref[:]
    ones_row = jnp.ones((1, D), f32)
    ann = g_ref[0:AW, :]
    an_row = jnp.sqrt(lax.dot_general(
        ones_row, ann * ann, (((1,), (1,)), ((), ())),
        preferred_element_type=f32))
    m = jnp.zeros((S, A), f32)
    swe = jnp.zeros((S, D), f32)
    for ws in range(W):
        sen_ws = g_ref[AW + ws * S:AW + (ws + 1) * S, :]
        swe = swe + sen_ws
        sn_c = jnp.sqrt(jnp.sum(sen_ws * sen_ws, axis=1, keepdims=True))
        eqcnt = jnp.sum(jnp.where(sen_ws == none_row, 1.0, 0.0),
                        axis=1, keepdims=True)
        isnone_c = eqcnt == f32(D)
        d_ws = lax.dot_general(sen_ws, ann, (((1,), (1,)), ((), ())),
                               preferred_element_type=f32)
        for wa in range(W):
            dvals = d_ws[:, wa * A:(wa + 1) * A]
            an_r = an_row[:, wa * A:(wa + 1) * A]
            denom = jnp.maximum(sn_c * an_r, 1e-8)
            sval = dvals / denom
            sval = jnp.where(isnone_c, 0.0, sval)
            m = jnp.where((sval >= m) | (sval < 0.0), sval, m)
    sims_ref[:] = m
    swe_ref[:] = swe


_dense = pl.pallas_call(
    _dense_body,
    out_shape=[
        jax.ShapeDtypeStruct((S, A), jnp.float32),
        jax.ShapeDtypeStruct((S, D), jnp.float32),
    ],
)


def kernel(ann_cats, sen_cats, none_idx, table):
    ann_t = ann_cats.astype(jnp.int32).T.reshape(-1)
    sen_t = sen_cats.astype(jnp.int32).T.reshape(-1)
    idx = jnp.concatenate([ann_t, sen_t], axis=0)
    g, none_row = _get_gather()(idx, none_idx.astype(jnp.int32), table)
    max_sims, swe = _dense(g, none_row)
    return max_sims, swe
